# Initial kernel scaffold; baseline (speedup 1.0000x reference)
import jax
import jax.numpy as jnp
from jax.experimental import pallas as pl


def kernel(x_author, x_paper, x_subject, edge_writes, edge_written_by, edge_about, edge_includes, params):
    raise NotImplementedError("write your pallas kernel here")


# speedup vs baseline: 1.0009x; 1.0009x over previous
import jax
import jax.numpy as jnp
from jax.experimental import pallas as pl

_RELS = [("author", "writes", "paper"), ("paper", "written_by", "author"),
         ("paper", "about", "subject"), ("subject", "includes", "paper")]
_N = {"author": 50000, "paper": 50000, "subject": 10000}


def _gat(x_src, x_dst, ei, p, num_dst):
    h_s = x_src @ p["W_src"]
    h_d = x_dst @ p["W_dst"]
    a_s = h_s @ p["att_src"]
    a_d = h_d @ p["att_dst"]
    src, dst = ei[0], ei[1]
    e = jax.nn.leaky_relu(a_s[src] + a_d[dst], negative_slope=0.2)
    m = jax.ops.segment_max(e, dst, num_segments=num_dst)
    w = jnp.exp(e - m[dst])
    z = jax.ops.segment_sum(w, dst, num_segments=num_dst)
    coef = w / (z[dst] + 1e-16)
    out = jax.ops.segment_sum(coef[:, None] * h_s[src], dst, num_segments=num_dst)
    return out + p["b"]


def _hetero(xd, edges, layer_p):
    acc = {}
    for (s, name, d) in _RELS:
        out = _gat(xd[s], xd[d], edges[name], layer_p[name], _N[d])
        acc.setdefault(d, []).append(out)
    return {k: sum(v) / float(len(v)) for k, v in acc.items()}


def _ln_kernel(x_ref, g_ref, b_ref, o_ref):
    x = x_ref[...]
    mu = jnp.mean(x, axis=-1, keepdims=True)
    var = jnp.mean((x - mu) ** 2, axis=-1, keepdims=True)
    o_ref[...] = (x - mu) / jnp.sqrt(var + 1e-5) * g_ref[...] + b_ref[...]


def _ln(x, g, b):
    n, d = x.shape
    blk = 2048
    npad = ((n + blk - 1) // blk) * blk
    xp = jnp.pad(x, ((0, npad - n), (0, 0)))
    out = pl.pallas_call(
        _ln_kernel,
        grid=(npad // blk,),
        in_specs=[pl.BlockSpec((blk, d), lambda i: (i, 0)),
                  pl.BlockSpec((d,), lambda i: (0,)),
                  pl.BlockSpec((d,), lambda i: (0,))],
        out_specs=pl.BlockSpec((blk, d), lambda i: (i, 0)),
        out_shape=jax.ShapeDtypeStruct((npad, d), x.dtype),
    )(xp, g, b)
    return out[:n]


def kernel(x_author, x_paper, x_subject, edge_writes, edge_written_by, edge_about, edge_includes, params):
    xd = {"author": x_author, "paper": x_paper, "subject": x_subject}
    edges = {"writes": edge_writes, "written_by": edge_written_by,
             "about": edge_about, "includes": edge_includes}
    xd = _hetero(xd, edges, params["conv1"])
    xd = {k: jax.nn.elu(v) for k, v in xd.items()}
    xd = _hetero(xd, edges, params["conv2"])
    xd = {k: _ln(v, params["ln_g"], params["ln_b"]) for k, v in xd.items()}
    return (xd["author"], xd["paper"], xd["subject"])


# speedup vs baseline: 1.8441x; 1.8425x over previous
import dataclasses
import functools

import jax
import jax.numpy as jnp
from jax import lax
from jax.experimental import pallas as pl
from jax.experimental.pallas import tpu as pltpu
from jax.experimental.pallas import tpu_sc as plsc

_N = {"author": 50000, "paper": 50000, "subject": 10000}
_RELS = [("author", "writes", "paper"), ("paper", "written_by", "author"),
         ("paper", "about", "subject"), ("subject", "includes", "paper")]
_E = 150000
_D = 128
_DX = 144
_CHUNK = 512
_EC = 128
_NTILES = 32
_LANES = 16

_mesh = plsc.VectorSubcoreMesh(core_axis_name="c", subcore_axis_name="s",
                               num_cores=2, num_subcores=16)

_sc_params = pltpu.CompilerParams()
if "needs_layout_passes" in pltpu.CompilerParams.__dataclass_fields__:
    _sc_params = dataclasses.replace(_sc_params, needs_layout_passes=False)


def _rel_cfg(ns, nd):
    nd_pad = -(-nd // (_NTILES * _CHUNK)) * (_NTILES * _CHUNK)
    nch = nd_pad // _CHUNK
    e_pad = -(-(_E + _EC * nch) // (_EC * _NTILES)) * (_EC * _NTILES)
    return dict(NS=ns, ND=nd, ND_PAD=nd_pad, NCH=nch, E_PAD=e_pad)


_CFG = {name: _rel_cfg(_N[s], _N[d]) for (s, name, d) in _RELS}


def _prep_edges(ei, cfg):
    nch, e_pad, nd = cfg["NCH"], cfg["E_PAD"], cfg["ND"]
    dst_s, src_s = lax.sort((ei[1].astype(jnp.int32), ei[0].astype(jnp.int32)),
                            num_keys=1)
    bounds = jnp.arange(nch + 1, dtype=jnp.int32) * _CHUNK
    ptr = jnp.searchsorted(dst_s, bounds).astype(jnp.int32)
    cnt = ptr[1:] - ptr[:-1]
    pcnt = -(-cnt // _EC) * _EC
    pe_ptr = jnp.concatenate([jnp.zeros((1,), jnp.int32),
                              jnp.cumsum(pcnt)]).astype(jnp.int32)
    chunk_id = dst_s // _CHUNK
    pos = pe_ptr[chunk_id] + (jnp.arange(_E, dtype=jnp.int32) - ptr[chunk_id])
    src_p = jnp.zeros((e_pad,), jnp.int32).at[pos].set(src_s)
    dst_p = jnp.zeros((e_pad,), jnp.int32).at[pos].set(dst_s)
    dloc_p = jnp.full((e_pad,), _CHUNK, jnp.int32).at[pos].set(dst_s - chunk_id * _CHUNK)
    ecp = jnp.zeros((nch + 16,), jnp.int32).at[:nch + 1].set(pe_ptr // _EC)
    return dict(src=src_p, dst=dst_p, dloc=dloc_p, ecp=ecp)


def _proj_body(x_ref, w_ref, o_ref):
    o_ref[...] = jnp.dot(x_ref[...], w_ref[...], preferred_element_type=jnp.float32)


def _tc_proj(x, w144):
    n = x.shape[0]
    blk = 400
    return pl.pallas_call(
        _proj_body,
        grid=(n // blk,),
        in_specs=[pl.BlockSpec((blk, _D), lambda i: (i, 0)),
                  pl.BlockSpec((_D, _D), lambda i: (0, 0))],
        out_specs=pl.BlockSpec((blk, _D), lambda i: (i, 0)),
        out_shape=jax.ShapeDtypeStruct((n, _D), jnp.float32),
    )(x, w144)


def _att_body(x_ref, v_ref, o_ref):
    o_ref[...] = jnp.dot(x_ref[...], v_ref[...], preferred_element_type=jnp.float32)


def _tc_att(x, vmat):
    n = x.shape[0]
    blk = 400
    k = vmat.shape[1]
    return pl.pallas_call(
        _att_body,
        grid=(n // blk,),
        in_specs=[pl.BlockSpec((blk, _D), lambda i: (i, 0)),
                  pl.BlockSpec((_D, k), lambda i: (0, 0))],
        out_specs=pl.BlockSpec((blk, k), lambda i: (i, 0)),
        out_shape=jax.ShapeDtypeStruct((n, k), jnp.float32),
    )(x, vmat)


def _finish(a_ref, z_ref, b_ref):
    return a_ref[...] / (z_ref[:, :1] + 1e-16) + b_ref[...]


def _post1_body(a_ref, z_ref, b_ref, o_ref):
    v = _finish(a_ref, z_ref, b_ref)
    o_ref[...] = jnp.where(v > 0, v, (jnp.exp(v) - 1.0))


def _post2_body(a_ref, z_ref, b_ref, a2_ref, z2_ref, b2_ref, o_ref):
    v = 0.5 * (_finish(a_ref, z_ref, b_ref) + _finish(a2_ref, z2_ref, b2_ref))
    o_ref[...] = jnp.where(v > 0, v, (jnp.exp(v) - 1.0))


def _ln1_body(a_ref, z_ref, b_ref, g_ref, bb_ref, o_ref):
    o_ref[...] = _ln_rows(_finish(a_ref, z_ref, b_ref), g_ref, bb_ref)


def _ln2_body(a_ref, z_ref, b_ref, a2_ref, z2_ref, b2_ref, g_ref, bb_ref, o_ref):
    v = 0.5 * (_finish(a_ref, z_ref, b_ref) + _finish(a2_ref, z2_ref, b2_ref))
    o_ref[...] = _ln_rows(v, g_ref, bb_ref)


def _ln_rows(v, g_ref, bb_ref):
    mu = jnp.mean(v, axis=-1, keepdims=True)
    var = jnp.mean((v - mu) ** 2, axis=-1, keepdims=True)
    return (v - mu) / jnp.sqrt(var + 1e-5) * g_ref[...] + bb_ref[...]


def _tc_post(body, aggs, vecs, n_pad):
    blk = 512
    args = []
    in_specs = []
    for (a, z), b in zip(aggs, vecs["biases"]):
        in_specs += [pl.BlockSpec((blk, _D), lambda i: (i, 0)),
                     pl.BlockSpec((blk, 16), lambda i: (i, 0)),
                     pl.BlockSpec((_D,), lambda i: (0,))]
        args += [a, z, b]
    for extra in vecs.get("ln", []):
        in_specs.append(pl.BlockSpec((_D,), lambda i: (0,)))
        args.append(extra)
    return pl.pallas_call(
        body,
        grid=(n_pad // blk,),
        in_specs=in_specs,
        out_specs=pl.BlockSpec((blk, _D), lambda i: (i, 0)),
        out_shape=jax.ShapeDtypeStruct((n_pad, _D), jnp.float32),
    )(*args)


def _sc_w(a_s, a_d, src_p, dst_p, e_pad):
    ns, nd = a_s.shape[0], a_d.shape[0]
    nck = e_pad // _EC // _NTILES

    @functools.partial(
        pl.kernel, mesh=_mesh,
        out_type=jax.ShapeDtypeStruct((e_pad,), jnp.float32),
        scratch_types=[pltpu.VMEM((ns,), jnp.float32),
                       pltpu.VMEM((nd,), jnp.float32),
                       pltpu.VMEM((_EC,), jnp.int32),
                       pltpu.VMEM((_EC,), jnp.int32),
                       pltpu.VMEM((_EC,), jnp.float32),
                       pltpu.SemaphoreType.DMA],
        compiler_params=_sc_params,
    )
    def k(as_hbm, ad_hbm, src_hbm, dst_hbm, w_hbm, asv, adv, sidx, didx, wv, sem):
        wid = lax.axis_index("s") * 2 + lax.axis_index("c")
        pltpu.sync_copy(as_hbm, asv)
        pltpu.sync_copy(ad_hbm, adv)

        @pl.loop(0, nck)
        def _(i):
            base = (wid * nck + i) * _EC
            pltpu.sync_copy(src_hbm.at[pl.ds(base, _EC)], sidx)
            pltpu.sync_copy(dst_hbm.at[pl.ds(base, _EC)], didx)
            for g in range(_EC // _LANES):
                sl = pl.ds(g * _LANES, _LANES)
                e = (plsc.load_gather(asv, [sidx[sl]])
                     + plsc.load_gather(adv, [didx[sl]]))
                e = jnp.maximum(e, 0.2 * e)
                wv[sl] = jnp.exp(e)
            pltpu.sync_copy(wv, w_hbm.at[pl.ds(base, _EC)])

    return k(a_s, a_d, src_p, dst_p)


def _sc_agg(h, w_p, src_p, dloc_p, ecp, cfg):
    nch, nd_pad = cfg["NCH"], cfg["ND_PAD"]
    cpt = nch // _NTILES
    outb_n = (_CHUNK + 1) * _D
    zb_n = (_CHUNK + 1) * _LANES

    @functools.partial(
        pl.kernel, mesh=_mesh,
        out_type=(jax.ShapeDtypeStruct((nd_pad * _D,), jnp.float32),
                  jax.ShapeDtypeStruct((nd_pad * _LANES,), jnp.float32)),
        scratch_types=[pltpu.VMEM((outb_n,), jnp.float32),
                       pltpu.VMEM((zb_n,), jnp.float32),
                       pltpu.VMEM((_EC, _D), jnp.float32),
                       pltpu.VMEM((_EC,), jnp.int32),
                       pltpu.VMEM((_EC,), jnp.int32),
                       pltpu.VMEM((_EC,), jnp.float32),
                       pltpu.VMEM((nch + 16,), jnp.int32),
                       pltpu.SemaphoreType.DMA],
        compiler_params=_sc_params,
    )
    def k(h_hbm, w_hbm, src_hbm, dloc_hbm, ecp_hbm, out_hbm, z_hbm,
          outb, zb, gb, sidx, dlv, wv, ecpv, sem):
        wid = lax.axis_index("s") * 2 + lax.axis_index("c")
        pltpu.sync_copy(ecp_hbm, ecpv)
        lane0 = (lax.broadcasted_iota(jnp.int32, (_LANES,), 0) == 0
                 ).astype(jnp.float32)

        @pl.loop(0, cpt)
        def _(ci):
            c = wid * cpt + ci

            @pl.loop(0, outb_n // _LANES)
            def _(z):
                outb[pl.ds(z * _LANES, _LANES)] = jnp.zeros((_LANES,), jnp.float32)

            @pl.loop(0, zb_n // _LANES)
            def _(z):
                zb[pl.ds(z * _LANES, _LANES)] = jnp.zeros((_LANES,), jnp.float32)

            kv = plsc.load_gather(
                ecpv, [lax.broadcasted_iota(jnp.int32, (_LANES,), 0) + c])
            k0 = kv[0]
            k1 = kv[1]

            @pl.loop(k0, k1)
            def _(kk):
                base = kk * _EC
                pltpu.sync_copy(src_hbm.at[pl.ds(base, _EC)], sidx)
                pltpu.sync_copy(dloc_hbm.at[pl.ds(base, _EC)], dlv)
                pltpu.sync_copy(w_hbm.at[pl.ds(base, _EC)], wv)
                pltpu.async_copy(h_hbm.at[sidx], gb, sem).wait()

                @pl.loop(0, _EC // _LANES)
                def _(g):
                    d16 = dlv[pl.ds(g * _LANES, _LANES)]
                    w16 = wv[pl.ds(g * _LANES, _LANES)]
                    doff = d16 * _D
                    for e in range(_LANES):
                        off = doff[e]
                        wj = w16[e]
                        plsc.addupdate(zb.at[pl.ds(d16[e] * _LANES, _LANES)],
                                       wj * lane0)
                        for f in range(_D // _LANES):
                            sl = pl.ds(f * _LANES, _LANES)
                            plsc.addupdate(outb.at[pl.ds(off + f * _LANES, _LANES)],
                                           wj * gb[g * _LANES + e, sl])

            pltpu.sync_copy(outb.at[pl.ds(0, _CHUNK * _D)],
                            out_hbm.at[pl.ds(c * _CHUNK * _D, _CHUNK * _D)])
            pltpu.sync_copy(zb.at[pl.ds(0, _CHUNK * _LANES)],
                            z_hbm.at[pl.ds(c * _CHUNK * _LANES, _CHUNK * _LANES)])

    out, z = k(h, w_p, src_p, dloc_p, ecp)
    return out.reshape(nd_pad, _D), z.reshape(nd_pad, _LANES)


def _layer(xd, pre, layer_p, final_ln, ln_g, ln_b):
    type_cols = {"author": [], "paper": [], "subject": []}
    for (s, name, d) in _RELS:
        type_cols[s].append((name, "src", layer_p[name]["W_src"] @ layer_p[name]["att_src"]))
        type_cols[d].append((name, "dst", layer_p[name]["W_dst"] @ layer_p[name]["att_dst"]))
    a_of = {}
    for t, cols in type_cols.items():
        vmat = jnp.stack([v for (_, _, v) in cols], axis=1)
        vmat = jnp.pad(vmat, ((0, 0), (0, 8 - vmat.shape[1])))
        at = _tc_att(xd[t], vmat)
        for j, (name, role, _) in enumerate(cols):
            a_of[(name, role)] = at[:, j]

    agg = {}
    for (s, name, d) in _RELS:
        cfg = _CFG[name]
        h = _tc_proj(xd[s], layer_p[name]["W_src"])
        w = _sc_w(a_of[(name, "src")], a_of[(name, "dst")],
                  pre[name]["src"], pre[name]["dst"], cfg["E_PAD"])
        agg[name] = _sc_agg(h, w, pre[name]["src"], pre[name]["dloc"],
                            pre[name]["ecp"], cfg)

    out = {}
    b = {name: layer_p[name]["b"] for (_, name, _) in _RELS}
    if final_ln:
        out["author"] = _tc_post(_ln1_body, [agg["written_by"]],
                                 dict(biases=[b["written_by"]], ln=[ln_g, ln_b]),
                                 _CFG["written_by"]["ND_PAD"])
        out["subject"] = _tc_post(_ln1_body, [agg["about"]],
                                  dict(biases=[b["about"]], ln=[ln_g, ln_b]),
                                  _CFG["about"]["ND_PAD"])
        out["paper"] = _tc_post(_ln2_body, [agg["writes"], agg["includes"]],
                                dict(biases=[b["writes"], b["includes"]], ln=[ln_g, ln_b]),
                                _CFG["writes"]["ND_PAD"])
    else:
        out["author"] = _tc_post(_post1_body, [agg["written_by"]],
                                 dict(biases=[b["written_by"]]),
                                 _CFG["written_by"]["ND_PAD"])
        out["subject"] = _tc_post(_post1_body, [agg["about"]],
                                  dict(biases=[b["about"]]),
                                  _CFG["about"]["ND_PAD"])
        out["paper"] = _tc_post(_post2_body, [agg["writes"], agg["includes"]],
                                dict(biases=[b["writes"], b["includes"]]),
                                _CFG["writes"]["ND_PAD"])
    return {t: out[t][:_N[t]] for t in out}


def kernel(x_author, x_paper, x_subject, edge_writes, edge_written_by,
           edge_about, edge_includes, params):
    edges = {"writes": edge_writes, "written_by": edge_written_by,
             "about": edge_about, "includes": edge_includes}
    pre = {name: _prep_edges(edges[name], _CFG[name]) for (_, name, _) in _RELS}
    xd = {"author": x_author, "paper": x_paper, "subject": x_subject}
    xd = _layer(xd, pre, params["conv1"], False, None, None)
    xd = _layer(xd, pre, params["conv2"], True, params["ln_g"], params["ln_b"])
    return (xd["author"], xd["paper"], xd["subject"])
